# Initial kernel scaffold; baseline (speedup 1.0000x reference)
import jax
import jax.numpy as jnp
from jax.experimental import pallas as pl


def kernel(x, adj, W, b, W1, b1, W2, b2):
    raise NotImplementedError("write your pallas kernel here")


# speedup vs baseline: 1.0298x; 1.0298x over previous
import functools

import jax
import jax.numpy as jnp
from jax.experimental import pallas as pl
from jax.experimental.pallas import tpu as pltpu

N = 10000
F = 128

BM = 400


def _pre_kernel(x_ref, w_ref, b_ref, w1_ref, z_ref, s1_ref):
    xv = x_ref[...]
    z_ref[...] = jnp.dot(xv, w_ref[...], preferred_element_type=jnp.float32) + b_ref[...]
    s1_ref[...] = jnp.dot(xv, w1_ref[...], preferred_element_type=jnp.float32)


def _gc1_kernel(adj_ref, s1_ref, b1_ref, z_ref, x1_ref):
    g = jnp.dot(adj_ref[...], s1_ref[...], preferred_element_type=jnp.float32)
    x1_ref[...] = jnp.maximum(g + b1_ref[...], 0.0) + z_ref[...]


def _mid_kernel(x1_ref, w2_ref, s2_ref):
    s2_ref[...] = jnp.dot(x1_ref[...], w2_ref[...], preferred_element_type=jnp.float32)


def _gc2_kernel(adj_ref, s2_ref, b2_ref, out_ref):
    g = jnp.dot(adj_ref[...], s2_ref[...], preferred_element_type=jnp.float32)
    g = g + b2_ref[...]
    m = jnp.max(g, axis=1, keepdims=True)
    shifted = g - m
    lse = jnp.log(jnp.sum(jnp.exp(shifted), axis=1, keepdims=True))
    out_ref[...] = shifted - lse


@jax.jit
def _run(x, adj, W, b, W1, b1, W2, b2):
    b2d = b.reshape(1, F)
    b1_2d = b1.reshape(1, F)
    b2_2d = b2.reshape(1, F)

    z, s1 = pl.pallas_call(
        _pre_kernel,
        out_shape=(
            jax.ShapeDtypeStruct((N, F), jnp.float32),
            jax.ShapeDtypeStruct((N, F), jnp.float32),
        ),
    )(x, W, b2d, W1)

    grid = (N // BM,)
    row_spec = pl.BlockSpec((BM, F), lambda i: (i, 0))
    full_spec = pl.BlockSpec((N, F), lambda i: (0, 0))
    bias_spec = pl.BlockSpec((1, F), lambda i: (0, 0))
    adj_spec = pl.BlockSpec((BM, N), lambda i: (i, 0))

    x1 = pl.pallas_call(
        _gc1_kernel,
        grid=grid,
        in_specs=[adj_spec, full_spec, bias_spec, row_spec],
        out_specs=row_spec,
        out_shape=jax.ShapeDtypeStruct((N, F), jnp.float32),
        compiler_params=pltpu.CompilerParams(
            dimension_semantics=("parallel",),
        ),
    )(adj, s1, b1_2d, z)

    s2 = pl.pallas_call(
        _mid_kernel,
        out_shape=jax.ShapeDtypeStruct((N, F), jnp.float32),
    )(x1, W2)

    out = pl.pallas_call(
        _gc2_kernel,
        grid=grid,
        in_specs=[adj_spec, full_spec, bias_spec],
        out_specs=row_spec,
        out_shape=jax.ShapeDtypeStruct((N, F), jnp.float32),
        compiler_params=pltpu.CompilerParams(
            dimension_semantics=("parallel",),
        ),
    )(adj, s2, b2_2d)

    return out


def kernel(x, adj, W, b, W1, b1, W2, b2):
    return _run(x, adj, W, b, W1, b1, W2, b2)
